# Initial kernel scaffold; baseline (speedup 1.0000x reference)
import jax
import jax.numpy as jnp
from jax.experimental import pallas as pl


def kernel(x, table, W1, b1, W2, b2):
    raise NotImplementedError("write your pallas kernel here")


# speedup vs baseline: 2.8678x; 2.8678x over previous
import jax
import jax.numpy as jnp
from jax.experimental import pallas as pl
from jax.experimental.pallas import tpu as pltpu
from jax.experimental.pallas import tpu_sc as plsc

VOCAB = 1000000
EMB = 32
WIN = 5
HID = 256
OUT = 64
BATCH = 16384
N_IDX = BATCH * WIN
GWIN = 128
BB = 2048

_vector_mesh = plsc.VectorSubcoreMesh(
    core_axis_name="core", subcore_axis_name="subcore"
)


def _sc_gather(table, idx2d):
    @pl.kernel(
        out_type=jax.ShapeDtypeStruct((N_IDX, EMB), jnp.float32),
        mesh=_vector_mesh,
        compiler_params=pltpu.CompilerParams(use_tc_tiling_on_sc=False),
    )
    def gather_kernel(table_hbm, idx_hbm, out_hbm):
        def body(i_vmem, o_vmem):
            pltpu.sync_copy(table_hbm.at[i_vmem.at[0]], o_vmem)

        pltpu.emit_pipeline(
            body,
            grid=(N_IDX // GWIN,),
            in_specs=[pl.BlockSpec((1, GWIN), index_map=lambda i: (0, i))],
            out_specs=[pl.BlockSpec((GWIN, EMB), index_map=lambda i: (i, 0))],
            core_axis_name=("core", "subcore"),
            dimension_semantics=(pltpu.PARALLEL,),
        )(idx_hbm, out_hbm)

    return gather_kernel(table, idx2d)


def _mlp_body(e_ref, w1_ref, b1_ref, w2_ref, b2_ref, o_ref):
    h = jnp.tanh(
        jnp.dot(e_ref[...], w1_ref[...], preferred_element_type=jnp.float32)
        + b1_ref[...]
    )
    o_ref[...] = (
        jnp.dot(h, w2_ref[...], preferred_element_type=jnp.float32) + b2_ref[...]
    )


def _tc_mlp(embeds, W1, b1, W2, b2):
    return pl.pallas_call(
        _mlp_body,
        grid=(BATCH // BB,),
        in_specs=[
            pl.BlockSpec((BB, WIN * EMB), lambda i: (i, 0)),
            pl.BlockSpec((WIN * EMB, HID), lambda i: (0, 0)),
            pl.BlockSpec((1, HID), lambda i: (0, 0)),
            pl.BlockSpec((HID, OUT), lambda i: (0, 0)),
            pl.BlockSpec((1, OUT), lambda i: (0, 0)),
        ],
        out_specs=pl.BlockSpec((BB, OUT), lambda i: (i, 0)),
        out_shape=jax.ShapeDtypeStruct((BATCH, OUT), jnp.float32),
    )(embeds, W1, b1.reshape(1, HID), W2, b2.reshape(1, OUT))


@jax.jit
def kernel(x, table, W1, b1, W2, b2):
    idx2d = x.reshape(1, N_IDX).astype(jnp.int32)
    rows = _sc_gather(table, idx2d)
    embeds = rows.reshape(BATCH, WIN * EMB)
    return _tc_mlp(embeds, W1, b1, W2, b2)


# speedup vs baseline: 2.8967x; 1.0101x over previous
import jax
import jax.numpy as jnp
from jax.experimental import pallas as pl
from jax.experimental.pallas import tpu as pltpu
from jax.experimental.pallas import tpu_sc as plsc

VOCAB = 1000000
EMB = 32
WIN = 5
HID = 256
OUT = 64
BATCH = 16384
GWIN = 128
NJ = BATCH // GWIN
NW = 8
BB = 2048

_vector_mesh = plsc.VectorSubcoreMesh(
    core_axis_name="core", subcore_axis_name="subcore"
)


def _sc_gather(table, xT):
    @pl.kernel(
        out_type=jax.ShapeDtypeStruct((2, BATCH, 128), jnp.float32),
        mesh=_vector_mesh,
        compiler_params=pltpu.CompilerParams(use_tc_tiling_on_sc=False),
    )
    def gather_kernel(table_hbm, idx_hbm, out_hbm):
        def body(i_vmem, o_vmem):
            pltpu.sync_copy(table_hbm.at[i_vmem.at[0]], o_vmem.at[0])

        pltpu.emit_pipeline(
            body,
            grid=(NW, NJ),
            in_specs=[
                pl.BlockSpec(
                    (1, GWIN),
                    index_map=lambda w, j: (jnp.where(w < WIN, w, w - 4), j),
                )
            ],
            out_specs=[
                pl.BlockSpec(
                    (1, GWIN, EMB),
                    index_map=lambda w, j: (w // 4, j, w % 4),
                )
            ],
            core_axis_name=("core", "subcore"),
            dimension_semantics=(pltpu.PARALLEL, pltpu.PARALLEL),
        )(idx_hbm, out_hbm)

    return gather_kernel(table, xT)


def _mlp_body(e_ref, w1a_ref, w1b_ref, b1_ref, w2_ref, b2_ref, o_ref):
    h = jnp.tanh(
        jnp.dot(e_ref[0], w1a_ref[...], preferred_element_type=jnp.float32)
        + jnp.dot(e_ref[1], w1b_ref[...], preferred_element_type=jnp.float32)
        + b1_ref[...]
    )
    o_ref[...] = (
        jnp.dot(h, w2_ref[...], preferred_element_type=jnp.float32) + b2_ref[...]
    )


def _tc_mlp(eg, W1a, W1b, b1, W2, b2):
    return pl.pallas_call(
        _mlp_body,
        grid=(BATCH // BB,),
        in_specs=[
            pl.BlockSpec((2, BB, 128), lambda i: (0, i, 0)),
            pl.BlockSpec((128, HID), lambda i: (0, 0)),
            pl.BlockSpec((128, HID), lambda i: (0, 0)),
            pl.BlockSpec((1, HID), lambda i: (0, 0)),
            pl.BlockSpec((HID, OUT), lambda i: (0, 0)),
            pl.BlockSpec((1, OUT), lambda i: (0, 0)),
        ],
        out_specs=pl.BlockSpec((BB, OUT), lambda i: (i, 0)),
        out_shape=jax.ShapeDtypeStruct((BATCH, OUT), jnp.float32),
    )(eg, W1a, W1b, b1.reshape(1, HID), W2, b2.reshape(1, OUT))


@jax.jit
def kernel(x, table, W1, b1, W2, b2):
    xT = jnp.swapaxes(x, 0, 1).astype(jnp.int32)
    eg = _sc_gather(table, xT)
    W1a = W1[:128]
    W1b = jnp.zeros((128, HID), jnp.float32).at[: WIN * EMB - 128].set(W1[128:])
    return _tc_mlp(eg, W1a, W1b, b1, W2, b2)
